# baseline (speedup 1.0000x reference)
import functools

import jax
import jax.numpy as jnp
from jax.experimental import pallas as pl
from jax.experimental.pallas import tpu as pltpu

_C = 0.7978845608028654
_A = 0.044715


def _mlp_body(col_ref, x_ref, w1_ref, b1_ref, w2_ref, b2_ref, o_ref, h_ref, *,
              bf, nff):
    j = pl.program_id(1)
    x = x_ref[...].astype(jnp.bfloat16)
    h = jnp.dot(x, w1_ref[0].astype(jnp.bfloat16),
                preferred_element_type=jnp.float32)
    h = h + b1_ref[0, 0]
    m = h * h
    n = m * (2.0 * _C * _A) + (2.0 * _C)
    g = h * jax.nn.sigmoid(h * n)
    h_ref[:, pl.ds(j * bf, bf)] = g.astype(jnp.bfloat16)

    @pl.when(j == nff - 1)
    def _layer2():
        o_ref[...] = (
            jnp.dot(h_ref[...], w2_ref[0].astype(jnp.bfloat16),
                    preferred_element_type=jnp.float32)
            + b2_ref[0, 0]
        )


@functools.partial(jax.jit, static_argnames=("bt", "bf"))
def _moe_mlp(hidden_states, W1, b1, W2, b2, col, bt=1024, bf=1024):
    T, D = hidden_states.shape
    E, _, F = W1.shape
    nff = F // bf
    col_arr = jnp.atleast_1d(jnp.asarray(col, jnp.int32))
    b1r = b1.reshape(E, nff, 1, bf)
    b2r = b2.reshape(E, 1, 1, D)

    grid = (T // bt, nff)
    grid_spec = pltpu.PrefetchScalarGridSpec(
        num_scalar_prefetch=1,
        grid=grid,
        in_specs=[
            pl.BlockSpec((bt, D), lambda i, j, c: (i, 0)),
            pl.BlockSpec((1, D, bf), lambda i, j, c: (c[0], 0, j)),
            pl.BlockSpec((1, 1, 1, bf), lambda i, j, c: (c[0], j, 0, 0)),
            pl.BlockSpec((1, F, D), lambda i, j, c: (c[0], 0, 0)),
            pl.BlockSpec((1, 1, 1, D), lambda i, j, c: (c[0], 0, 0, 0)),
        ],
        out_specs=pl.BlockSpec((bt, D), lambda i, j, c: (i, 0)),
        scratch_shapes=[pltpu.VMEM((bt, F), jnp.bfloat16)],
    )
    body = functools.partial(_mlp_body, bf=bf, nff=nff)
    return pl.pallas_call(
        body,
        grid_spec=grid_spec,
        out_shape=jax.ShapeDtypeStruct((T, D), jnp.float32),
        compiler_params=pltpu.CompilerParams(
            dimension_semantics=("parallel", "arbitrary"),
        ),
    )(col_arr, hidden_states, W1, b1r, W2, b2r)


def kernel(hidden_states, W1, b1, W2, b2, col):
    return _moe_mlp(hidden_states, W1, b1, W2, b2, col)
